# Initial kernel scaffold; baseline (speedup 1.0000x reference)
import jax
import jax.numpy as jnp
from jax.experimental import pallas as pl


def kernel(logits_student, logits_teacher, labels, mask):
    raise NotImplementedError("write your pallas kernel here")


# speedup vs baseline: 66.2625x; 66.2625x over previous
import functools

import jax
import jax.numpy as jnp
from jax import lax
from jax.experimental import pallas as pl
from jax.experimental.pallas import tpu as pltpu
from jax.experimental.pallas import tpu_sc as plsc

B, L, V = 8, 32, 100000
K = 20
EPS = 1e-08
NEG = -1.0e30
ROWS = B * L
NW = 32
RPW = ROWS // NW
NVR = V // 16
NG = 10
NBPG = 25
NVPB = 25
NB = NG * NBPG
CAP = 1024
BIG = 1 << 30


def _v16(x, dtype):
    x = jnp.asarray(x)
    return x if x.shape == (16,) else jnp.full((16,), x, dtype)


def _scal(x):
    return jnp.max(x) if x.shape == (16,) else x


def _sc_body(t_hbm, s_hbm, tv_hbm, sv_hbm, st_hbm,
             buf, l1, cand_v, cand_i, outv, outi, outs, statv):
    wid = lax.axis_index("s") * 2 + lax.axis_index("c")
    io = lax.iota(jnp.int32, 16)
    zero16f = jnp.zeros((16,), jnp.float32)
    neg16 = jnp.full((16,), NEG, jnp.float32)

    outi[pl.ds(0, 16)] = jnp.zeros((16,), jnp.int32)
    outi[pl.ds(16, 16)] = jnp.zeros((16,), jnp.int32)

    def row_body(r, carry):
        row = wid * RPW + r
        pltpu.sync_copy(t_hbm.at[row], buf)

        l2 = []
        for g in range(NG):
            def blk_body(b, gmax, g=g):
                base = (g * NBPG + b) * (NVPB * 16)

                def v5(j, bm):
                    c = base + j * 80
                    x0 = buf[pl.ds(c, 16)]
                    x1 = buf[pl.ds(c + 16, 16)]
                    x2 = buf[pl.ds(c + 32, 16)]
                    x3 = buf[pl.ds(c + 48, 16)]
                    x4 = buf[pl.ds(c + 64, 16)]
                    m = jnp.maximum(jnp.maximum(x0, x1),
                                    jnp.maximum(jnp.maximum(x2, x3), x4))
                    return jnp.maximum(bm, m)

                bm = lax.fori_loop(0, NVPB // 5, v5, neg16)
                l1[pl.ds((g * NBPG + b) * 16, 16)] = bm
                return jnp.maximum(gmax, bm)

            l2.append(lax.fori_loop(0, NBPG, blk_body, neg16))

        gm = l2[0]
        for g in range(1, NG):
            gm = jnp.maximum(gm, l2[g])
        m_t = jnp.max(gm)

        l2s = list(l2)
        tau = m_t
        for _ in range(K):
            mx = l2s[0]
            for g in range(1, NG):
                mx = jnp.maximum(mx, l2s[g])
            tau = jnp.max(mx)
            tau_b = jnp.full((16,), tau, jnp.float32)
            l2s = [jnp.where(x >= tau_b, neg16, x) for x in l2s]
        tau_v = jnp.full((16,), tau, jnp.float32)

        m_tv = jnp.full((16,), m_t, jnp.float32)

        def se_body(j, accs):
            a0, a1, a2, a3, a4 = accs
            c = j * 80
            x0 = buf[pl.ds(c, 16)]
            x1 = buf[pl.ds(c + 16, 16)]
            x2 = buf[pl.ds(c + 32, 16)]
            x3 = buf[pl.ds(c + 48, 16)]
            x4 = buf[pl.ds(c + 64, 16)]
            return (a0 + jnp.exp(x0 - m_tv), a1 + jnp.exp(x1 - m_tv),
                    a2 + jnp.exp(x2 - m_tv), a3 + jnp.exp(x3 - m_tv),
                    a4 + jnp.exp(x4 - m_tv))

        accs = lax.fori_loop(0, NVR // 5, se_body,
                             (zero16f, zero16f, zero16f, zero16f, zero16f))
        z_t = _scal(jnp.sum(accs[0] + accs[1] + accs[2] + accs[3] + accs[4]))

        def clr_body(j, _):
            cand_v[pl.ds(j * 16, 16)] = neg16
            return 0
        lax.fori_loop(0, CAP // 16, clr_body, 0)

        def coll_body(b, off):
            bm = l1[pl.ds(b * 16, 16)]
            bmax = jnp.max(bm)

            def do_block(off):
                def cv_body(j, off):
                    jj = b * NVPB + j
                    x = buf[pl.ds(jj * 16, 16)]
                    msk = x >= tau_v
                    cnt = _v16(plsc.all_reduce_population_count(msk), jnp.int32)
                    pos = plsc.cumsum(jnp.where(msk, 1, 0).astype(jnp.int32)) - 1 + off
                    pos = jnp.minimum(pos, jnp.int32(CAP - 1))
                    plsc.store_scatter(cand_v, [pos], x, mask=msk)
                    plsc.store_scatter(cand_i, [pos], jj * 16 + io, mask=msk)
                    return off + cnt
                return lax.fori_loop(0, NVPB, cv_body, off)

            return lax.cond(bmax >= tau, do_block, lambda off: off, off)

        off = lax.fori_loop(0, NB, coll_body, jnp.zeros((16,), jnp.int32))
        ncv = jnp.minimum((jnp.max(off) + 15) // 16, jnp.int32(CAP // 16))

        lane0 = io == 0

        def ext_body(k, _):
            def smax_body(jj, mv):
                return jnp.maximum(mv, cand_v[pl.ds(jj * 16, 16)])
            mv = lax.fori_loop(0, ncv, smax_body, neg16)
            vk = jnp.max(mv)
            vk_v = jnp.full((16,), vk, jnp.float32)

            def spos_body(jj, best):
                x = cand_v[pl.ds(jj * 16, 16)]
                eq = x == vk_v
                cnt = _v16(plsc.all_reduce_population_count(eq), jnp.int32)
                ffs = _v16(plsc.all_reduce_ffs(eq), jnp.int32)
                pos = jj * 16 + ffs
                return jnp.minimum(best, jnp.where(cnt > 0, pos, BIG))

            best = lax.fori_loop(0, ncv, spos_body, jnp.full((16,), BIG, jnp.int32))
            best = jnp.minimum(best, jnp.int32(CAP - 1))
            iv = plsc.load_gather(cand_i, [best])
            kv = jnp.full((16,), k, jnp.int32)
            plsc.store_scatter(outv, [kv], vk_v, mask=lane0)
            plsc.store_scatter(outi, [kv], iv, mask=lane0)
            plsc.store_scatter(cand_v, [best], neg16, mask=lane0)
            return 0

        lax.fori_loop(0, K, ext_body, 0)
        pad_hi = jnp.where(io + 16 >= K, neg16, outv[pl.ds(16, 16)])
        outv[pl.ds(16, 16)] = pad_hi

        pltpu.sync_copy(s_hbm.at[row], buf)

        def sm_body(j, bm):
            c = j * 80
            x0 = buf[pl.ds(c, 16)]
            x1 = buf[pl.ds(c + 16, 16)]
            x2 = buf[pl.ds(c + 32, 16)]
            x3 = buf[pl.ds(c + 48, 16)]
            x4 = buf[pl.ds(c + 64, 16)]
            m = jnp.maximum(jnp.maximum(x0, x1),
                            jnp.maximum(jnp.maximum(x2, x3), x4))
            return jnp.maximum(bm, m)

        m_s = jnp.max(lax.fori_loop(0, NVR // 5, sm_body, neg16))
        m_sv = jnp.full((16,), m_s, jnp.float32)

        def se2_body(j, accs):
            a0, a1, a2, a3, a4 = accs
            c = j * 80
            x0 = buf[pl.ds(c, 16)]
            x1 = buf[pl.ds(c + 16, 16)]
            x2 = buf[pl.ds(c + 32, 16)]
            x3 = buf[pl.ds(c + 48, 16)]
            x4 = buf[pl.ds(c + 64, 16)]
            return (a0 + jnp.exp(x0 - m_sv), a1 + jnp.exp(x1 - m_sv),
                    a2 + jnp.exp(x2 - m_sv), a3 + jnp.exp(x3 - m_sv),
                    a4 + jnp.exp(x4 - m_sv))

        accs2 = lax.fori_loop(0, NVR // 5, se2_body,
                              (zero16f, zero16f, zero16f, zero16f, zero16f))
        z_s = _scal(jnp.sum(accs2[0] + accs2[1] + accs2[2] + accs2[3] + accs2[4]))

        iv0 = outi[pl.ds(0, 16)]
        sv0 = plsc.load_gather(buf, [iv0])
        outs[pl.ds(0, 16)] = sv0
        iv1 = outi[pl.ds(16, 16)]
        sv1 = plsc.load_gather(buf, [iv1])
        outs[pl.ds(16, 16)] = jnp.where(io + 16 < K, sv1, neg16)

        st = jnp.where(io == 0, jnp.full((16,), m_t, jnp.float32),
             jnp.where(io == 1, jnp.full((16,), z_t, jnp.float32),
             jnp.where(io == 2, jnp.full((16,), m_s, jnp.float32),
             jnp.where(io == 3, jnp.full((16,), z_s, jnp.float32), zero16f))))
        statv[pl.ds(0, 16)] = st

        pltpu.sync_copy(outv, tv_hbm.at[row])
        pltpu.sync_copy(outs, sv_hbm.at[row])
        pltpu.sync_copy(statv, st_hbm.at[row])
        return carry

    lax.fori_loop(0, RPW, row_body, 0)


@functools.partial(jax.jit, static_argnames=())
def _sc_call(t2, s2):
    mesh = plsc.VectorSubcoreMesh(core_axis_name="c", subcore_axis_name="s")
    f = pl.kernel(
        _sc_body,
        mesh=mesh,
        compiler_params=pltpu.CompilerParams(needs_layout_passes=False),
        out_type=[
            jax.ShapeDtypeStruct((ROWS, 32), jnp.float32),
            jax.ShapeDtypeStruct((ROWS, 32), jnp.float32),
            jax.ShapeDtypeStruct((ROWS, 16), jnp.float32),
        ],
        scratch_types=[
            pltpu.VMEM((V,), jnp.float32),
            pltpu.VMEM((NB * 16,), jnp.float32),
            pltpu.VMEM((CAP,), jnp.float32),
            pltpu.VMEM((CAP,), jnp.int32),
            pltpu.VMEM((32,), jnp.float32),
            pltpu.VMEM((32,), jnp.int32),
            pltpu.VMEM((32,), jnp.float32),
            pltpu.VMEM((16,), jnp.float32),
        ],
    )
    return f(t2, s2)


def _combine_body(tv_ref, sv_ref, st_ref, mk_ref, out_ref):
    tv = tv_ref[...]
    sv = sv_ref[...]
    m_t = st_ref[:, 0:1]
    z_t = st_ref[:, 1:2]
    m_s = st_ref[:, 2:3]
    z_s = st_ref[:, 3:4]
    pt = jnp.exp(tv - m_t) / z_t
    ps = jnp.exp(sv - m_s) / z_s
    sum_pt = jnp.sum(pt, axis=1, keepdims=True)
    sum_ps = jnp.sum(ps, axis=1, keepdims=True)
    alpha = sum_pt + EPS
    beta = sum_ps + EPS
    ptn = pt / alpha
    psn = ps / beta
    lr = jnp.log(jnp.maximum(ptn, EPS)) - jnp.log(jnp.maximum(psn, EPS))
    klt = jnp.sum(ptn * lr, axis=1, keepdims=True)
    at = 1.0 - sum_pt + EPS
    bs = 1.0 - sum_ps + EPS
    klq = at * jnp.log(jnp.maximum(at / bs, EPS))
    kl = (klt + klq) * mk_ref[...]
    out_ref[...] = (jnp.sum(kl) / B).reshape(1, 1)


def _combine_call(tv, sv, st, mk):
    return pl.pallas_call(
        _combine_body,
        out_shape=jax.ShapeDtypeStruct((1, 1), jnp.float32),
    )(tv, sv, st, mk)


def kernel(logits_student, logits_teacher, labels, mask):
    t2 = logits_teacher.reshape(ROWS, V)
    s2 = logits_student.reshape(ROWS, V)
    tv, sv, st = _sc_call(t2, s2)
    mk = mask.reshape(ROWS, 1).astype(jnp.float32)
    out = _combine_call(tv, sv, st, mk)
    return out.reshape(())
